# baseline (speedup 1.0000x reference)
import numpy as np

import jax
import jax.numpy as jnp
from jax.experimental import pallas as pl
from jax.experimental.pallas import tpu as pltpu

B, C_IN, H, W = 64, 512, 28, 28
HW = H * W
C_MID, GROWTH = 128, 32
C_OUT = C_IN + GROWTH
EPS = 1e-5
QMAX = 255.0
NUM_CHUNKS = 16


def _quantize_small(x):
    mn = x.min()
    mx = x.max()
    scale = jnp.maximum((mx - mn) / QMAX, 1e-8)
    q = jnp.round((jnp.clip(x, mn, mx) - mn) / scale)
    return q * scale + mn


def _stats_kernel(x_ref, s_ref, ss_ref, mn_ref, mx_ref):
    xb = x_ref[0]
    s_ref[0, 0] = jnp.sum(xb, axis=1)
    ss_ref[0, 0] = jnp.sum(xb * xb, axis=1)
    mn_ref[0, 0] = jnp.min(xb, axis=1)
    mx_ref[0, 0] = jnp.max(xb, axis=1)


def _conv1_kernel(x_ref, a_ref, t_ref, w1_ref, q_ref,
                  h2_ref, s_ref, mn_ref, mx_ref):
    xb = x_ref[0]
    a = a_ref[...]
    t = t_ref[...]
    h1 = jnp.maximum(xb * a + t, 0.0)
    inv1, mn1, sc1 = q_ref[0], q_ref[1], q_ref[2]
    q1 = jnp.clip(jnp.round((h1 - mn1) * inv1), 0.0, QMAX)
    xq = (q1 * sc1 + mn1).astype(jnp.bfloat16)
    h2 = jnp.dot(w1_ref[...], xq,
                 preferred_element_type=jnp.float32)
    h2b = h2.astype(jnp.bfloat16)
    h2_ref[0] = h2b
    h2f = h2b.astype(jnp.float32)
    s_ref[0, 0] = jnp.sum(h2f, axis=1)
    mn_ref[0, 0] = jnp.min(h2f, axis=1)
    mx_ref[0, 0] = jnp.max(h2f, axis=1)


_OFFS = [(ky - 1, kx - 1) for ky in range(3) for kx in range(3)]


def _conv2_kernel(x_ref, h2_ref, a_ref, c_ref, w2_ref, q_ref,
                  o_ref, f_scr, g_scr):
    h2 = h2_ref[0].astype(jnp.float32)
    h3 = jnp.maximum(h2 * a_ref[...] + c_ref[...], 0.0)
    inv3, mn3, sc3 = q_ref[0], q_ref[1], q_ref[2]
    q3 = jnp.clip(jnp.round((h3 - mn3) * inv3), 0.0, QMAX)
    xq3 = (q3 * sc3 + mn3).astype(jnp.bfloat16)

    f_scr[...] = jnp.zeros_like(f_scr)
    f_scr[:, 32:816] = xq3

    xcol = jax.lax.broadcasted_iota(jnp.int32, (1, HW), 1) % W
    for i, (dy, dx) in enumerate(_OFFS):
        off = 32 + dy * W + dx
        blk = f_scr[:, off:off + HW]
        if dx == -1:
            blk = jnp.where(xcol == 0, jnp.bfloat16(0), blk)
        elif dx == 1:
            blk = jnp.where(xcol == W - 1, jnp.bfloat16(0), blk)
        g_scr[i * C_MID:(i + 1) * C_MID, :] = blk

    h4 = jnp.dot(w2_ref[...], g_scr[...],
                 preferred_element_type=jnp.float32)
    o_ref[0, :C_IN] = x_ref[0]
    o_ref[0, C_IN:] = h4


def kernel(x, bn1_w, bn1_b, conv1_w, rbn_w, rbn_b, conv2_w):
    xr = x.reshape(B, C_IN, HW)

    stat_shape = jax.ShapeDtypeStruct((B, 1, C_IN), jnp.float32)
    stat_spec = pl.BlockSpec((1, 1, C_IN), lambda b: (b, 0, 0))
    sums, sumsqs, mins, maxs = pl.pallas_call(
        _stats_kernel,
        grid=(B,),
        in_specs=[pl.BlockSpec((1, C_IN, HW), lambda b: (b, 0, 0))],
        out_specs=[stat_spec] * 4,
        out_shape=[stat_shape] * 4,
        compiler_params=pltpu.CompilerParams(
            dimension_semantics=("arbitrary",)),
        name="dense_stats",
    )(xr)

    n1 = float(B * HW)
    mean1 = jnp.sum(sums[:, 0], axis=0) / n1
    var1 = jnp.sum(sumsqs[:, 0], axis=0) / n1 - mean1 * mean1
    a1 = bn1_w * jax.lax.rsqrt(var1 + EPS)
    t1 = bn1_b - mean1 * a1
    cmin = jnp.min(mins[:, 0], axis=0)
    cmax = jnp.max(maxs[:, 0], axis=0)
    lo = a1 * cmin + t1
    hi = a1 * cmax + t1
    mn1 = jnp.maximum(jnp.min(lo), 0.0)
    mx1 = jnp.maximum(jnp.max(hi), 0.0)
    sc1 = jnp.maximum((mx1 - mn1) / QMAX, 1e-8)
    qv1 = jnp.stack([1.0 / sc1, mn1, sc1])

    w1q = _quantize_small(conv1_w.reshape(C_MID, C_IN)).astype(jnp.bfloat16)

    mstat_shape = jax.ShapeDtypeStruct((B, 1, C_MID), jnp.float32)
    mstat_spec = pl.BlockSpec((1, 1, C_MID), lambda b: (b, 0, 0))
    vec_spec_in = pl.BlockSpec((C_IN, 1), lambda b: (0, 0))
    h2, s2, mn2, mx2 = pl.pallas_call(
        _conv1_kernel,
        grid=(B,),
        in_specs=[
            pl.BlockSpec((1, C_IN, HW), lambda b: (b, 0, 0)),
            vec_spec_in, vec_spec_in,
            pl.BlockSpec((C_MID, C_IN), lambda b: (0, 0)),
            pl.BlockSpec(memory_space=pltpu.SMEM),
        ],
        out_specs=[
            pl.BlockSpec((1, C_MID, HW), lambda b: (b, 0, 0)),
            mstat_spec, mstat_spec, mstat_spec,
        ],
        out_shape=[
            jax.ShapeDtypeStruct((B, C_MID, HW), jnp.bfloat16),
            mstat_shape, mstat_shape, mstat_shape,
        ],
        compiler_params=pltpu.CompilerParams(
            dimension_semantics=("arbitrary",)),
        name="dense_conv1",
    )(xr, a1[:, None], t1[:, None], w1q, qv1)

    mean2 = jnp.sum(s2[:, 0], axis=0) / n1
    ch_mx = jnp.max(mx2[:, 0].reshape(NUM_CHUNKS, 4, C_MID), axis=1)
    ch_mn = jnp.min(mn2[:, 0].reshape(NUM_CHUNKS, 4, C_MID), axis=1)
    mean_max = jnp.mean(ch_mx, axis=0)
    mean_min = jnp.mean(ch_mn, axis=0)
    n_chunk = float(B * HW // NUM_CHUNKS)
    scale_fix = ((0.5 * 0.35) * (1.0 + (np.pi * np.log(4.0)) ** 0.5)
                 / ((2.0 * np.log(n_chunk)) ** 0.5))
    scale2 = 1.0 / ((mean_max - mean_min) * scale_fix + EPS)
    qw2 = _quantize_small(rbn_w)
    qb2 = _quantize_small(rbn_b)
    a2 = scale2 * qw2
    c2 = qb2 - mean2 * a2

    gmin2 = jnp.min(mn2[:, 0], axis=0)
    gmax2 = jnp.max(mx2[:, 0], axis=0)
    lo3 = a2 * gmin2 + c2
    hi3 = a2 * gmax2 + c2
    mn3 = jnp.maximum(jnp.min(lo3), 0.0)
    mx3 = jnp.maximum(jnp.max(hi3), 0.0)
    sc3 = jnp.maximum((mx3 - mn3) / QMAX, 1e-8)
    qv3 = jnp.stack([1.0 / sc3, mn3, sc3])

    w2q = _quantize_small(conv2_w)
    w2mat = w2q.transpose(0, 2, 3, 1).reshape(
        GROWTH, 9 * C_MID).astype(jnp.bfloat16)

    vec_spec_mid = pl.BlockSpec((C_MID, 1), lambda b: (0, 0))
    out = pl.pallas_call(
        _conv2_kernel,
        grid=(B,),
        in_specs=[
            pl.BlockSpec((1, C_IN, HW), lambda b: (b, 0, 0)),
            pl.BlockSpec((1, C_MID, HW), lambda b: (b, 0, 0)),
            vec_spec_mid, vec_spec_mid,
            pl.BlockSpec((GROWTH, 9 * C_MID), lambda b: (0, 0)),
            pl.BlockSpec(memory_space=pltpu.SMEM),
        ],
        out_specs=pl.BlockSpec((1, C_OUT, HW), lambda b: (b, 0, 0)),
        out_shape=jax.ShapeDtypeStruct((B, C_OUT, HW), jnp.float32),
        scratch_shapes=[
            pltpu.VMEM((C_MID, 896), jnp.bfloat16),
            pltpu.VMEM((9 * C_MID, HW), jnp.bfloat16),
        ],
        compiler_params=pltpu.CompilerParams(
            dimension_semantics=("arbitrary",)),
        name="dense_conv2",
    )(xr, h2, a2[:, None], c2[:, None], w2mat, qv3)

    return out.reshape(B, C_OUT, H, W)
